# baseline (speedup 1.0000x reference)
import functools

import jax
import jax.numpy as jnp
from jax import lax
from jax.experimental import pallas as pl
from jax.experimental.pallas import tpu as pltpu
from jax.experimental.pallas import tpu_sc as plsc

_DEG, _N, _D = 16, 10000, 256
_NC, _NS = 2, 16
_NW = _NC * _NS
_G = 8
_L = 16

_S = 1280
_NGRP = _S // _G
_NB = 80

_mesh = plsc.VectorSubcoreMesh(core_axis_name="c", subcore_axis_name="s")


@functools.partial(
    pl.kernel,
    out_type=jax.ShapeDtypeStruct((_S, _D), jnp.float32),
    mesh=_mesh,
    scratch_types=[
        pltpu.VMEM((_DEG, _G, _D), jnp.float32),
        pltpu.VMEM((_DEG, _G, _D), jnp.float32),
        pltpu.VMEM((_G, _D), jnp.float32),
        pltpu.VMEM((_G, _D), jnp.float32),
        pltpu.SemaphoreType.DMA,
        pltpu.SemaphoreType.DMA,
        pltpu.SemaphoreType.DMA,
        pltpu.SemaphoreType.DMA,
    ],
)
def _sc_sum(ft_hbm, out_hbm, in0, in1, ob0, ob1, si0, si1, so0, so1):
    wid = lax.axis_index("s") * _NC + lax.axis_index("c")
    g_lo = (_NGRP * wid) // _NW
    g_hi = (_NGRP * (wid + 1)) // _NW
    n_sub = g_hi - g_lo
    ins, obs = (in0, in1), (ob0, ob1)
    sis, sos = (si0, si1), (so0, so1)

    def in_dma(j, b):
        pltpu.async_copy(
            ft_hbm.at[:, pl.ds((g_lo + j) * _G, _G), :], ins[b], sis[b]
        )

    for b in range(2):
        in_dma(b, b)

    def step(j, b):
        pltpu.make_async_copy(
            ft_hbm.at[:, pl.ds(0, _G), :], ins[b], sis[b]
        ).wait()

        @pl.when(j >= 2)
        def _():
            pltpu.make_async_copy(
                obs[b], out_hbm.at[pl.ds(0, _G), :], sos[b]
            ).wait()

        def inner(i, carry):
            g = i // (_D // _L)
            c = (i % (_D // _L)) * _L
            acc = ins[b][0, g, pl.ds(c, _L)]
            for k in range(1, _DEG):
                acc = acc + ins[b][k, g, pl.ds(c, _L)]
            obs[b][g, pl.ds(c, _L)] = acc
            return carry

        lax.fori_loop(0, _G * _D // _L, inner, 0, unroll=4)

        @pl.when(j + 2 < n_sub)
        def _():
            in_dma(j + 2, b)

        pltpu.async_copy(
            obs[b], out_hbm.at[pl.ds((g_lo + j) * _G, _G), :], sos[b]
        )

    def body(j, carry):
        @pl.when(j % 2 == 0)
        def _():
            step(j, 0)

        @pl.when(j % 2 == 1)
        def _():
            step(j, 1)

        return carry

    lax.fori_loop(0, n_sub, body, 0)

    for b in range(2):
        pltpu.make_async_copy(
            obs[b], out_hbm.at[pl.ds(0, _G), :], sos[b]
        ).wait()


def _tc_body(ft_ref, out_ref):
    out_ref[...] = jnp.sum(ft_ref[...], axis=0)


def _tc_sum(ft):
    return pl.pallas_call(
        _tc_body,
        grid=((_N - _S) // _NB,),
        in_specs=[pl.BlockSpec((_DEG, _NB, _D), lambda i: (0, _S // _NB + i, 0))],
        out_specs=pl.BlockSpec((_NB, _D), lambda i: (i, 0)),
        out_shape=jax.ShapeDtypeStruct((_N - _S, _D), jnp.float32),
    )(ft)


def kernel(a, ft):
    del a
    out_sc = _sc_sum(ft)
    out_tc = _tc_sum(ft)
    return jnp.concatenate([out_sc, out_tc], axis=0)


# speedup vs baseline: 1.4467x; 1.4467x over previous
import functools

import jax
import jax.numpy as jnp
from jax import lax
from jax.experimental import pallas as pl
from jax.experimental.pallas import tpu as pltpu
from jax.experimental.pallas import tpu_sc as plsc

_DEG, _N, _D = 16, 10000, 256
_NC, _NS = 2, 16
_NW = _NC * _NS
_G = 8
_L = 16

_S = 2400
_NGRP = _S // _G
_NB = 400

_mesh = plsc.VectorSubcoreMesh(core_axis_name="c", subcore_axis_name="s")


@functools.partial(
    pl.kernel,
    out_type=jax.ShapeDtypeStruct((_S, _D), jnp.float32),
    mesh=_mesh,
    scratch_types=[
        pltpu.VMEM((_DEG, _G, _D), jnp.float32),
        pltpu.VMEM((_DEG, _G, _D), jnp.float32),
        pltpu.VMEM((_G, _D), jnp.float32),
        pltpu.VMEM((_G, _D), jnp.float32),
        pltpu.SemaphoreType.DMA,
        pltpu.SemaphoreType.DMA,
        pltpu.SemaphoreType.DMA,
        pltpu.SemaphoreType.DMA,
    ],
)
def _sc_sum(ft_hbm, out_hbm, in0, in1, ob0, ob1, si0, si1, so0, so1):
    wid = lax.axis_index("s") * _NC + lax.axis_index("c")
    g_lo = (_NGRP * wid) // _NW
    g_hi = (_NGRP * (wid + 1)) // _NW
    n_sub = g_hi - g_lo
    ins, obs = (in0, in1), (ob0, ob1)
    sis, sos = (si0, si1), (so0, so1)

    def in_dma(j, b):
        pltpu.async_copy(
            ft_hbm.at[:, pl.ds((g_lo + j) * _G, _G), :], ins[b], sis[b]
        )

    for b in range(2):
        in_dma(b, b)

    def step(j, b):
        pltpu.make_async_copy(
            ft_hbm.at[:, pl.ds(0, _G), :], ins[b], sis[b]
        ).wait()

        @pl.when(j >= 2)
        def _():
            pltpu.make_async_copy(
                obs[b], out_hbm.at[pl.ds(0, _G), :], sos[b]
            ).wait()

        def inner(i, carry):
            g = i // (_D // _L)
            c = (i % (_D // _L)) * _L
            acc = ins[b][0, g, pl.ds(c, _L)]
            for k in range(1, _DEG):
                acc = acc + ins[b][k, g, pl.ds(c, _L)]
            obs[b][g, pl.ds(c, _L)] = acc
            return carry

        lax.fori_loop(0, _G * _D // _L, inner, 0, unroll=4)

        @pl.when(j + 2 < n_sub)
        def _():
            in_dma(j + 2, b)

        pltpu.async_copy(
            obs[b], out_hbm.at[pl.ds((g_lo + j) * _G, _G), :], sos[b]
        )

    def body(j, carry):
        @pl.when(j % 2 == 0)
        def _():
            step(j, 0)

        @pl.when(j % 2 == 1)
        def _():
            step(j, 1)

        return carry

    lax.fori_loop(0, n_sub, body, 0)

    for b in range(2):
        pltpu.make_async_copy(
            obs[b], out_hbm.at[pl.ds(0, _G), :], sos[b]
        ).wait()


def _tc_body(ft_ref, out_ref):
    out_ref[...] = jnp.sum(ft_ref[...], axis=0)


def _tc_sum(ft):
    return pl.pallas_call(
        _tc_body,
        grid=((_N - _S) // _NB,),
        in_specs=[pl.BlockSpec((_DEG, _NB, _D), lambda i: (0, _S // _NB + i, 0))],
        out_specs=pl.BlockSpec((_NB, _D), lambda i: (i, 0)),
        out_shape=jax.ShapeDtypeStruct((_N - _S, _D), jnp.float32),
    )(ft)


def kernel(a, ft):
    del a
    out_sc = _sc_sum(ft)
    out_tc = _tc_sum(ft)
    return jnp.concatenate([out_sc, out_tc], axis=0)


# speedup vs baseline: 2.0959x; 1.4487x over previous
import jax
import jax.numpy as jnp
from jax.experimental import pallas as pl


_DEG, _N, _D = 16, 10000, 256
_NB = 1000


def _reduce_body(ft_ref, out_ref):
    out_ref[...] = jnp.sum(ft_ref[...], axis=0)


def kernel(a, ft):
    del a
    out = pl.pallas_call(
        _reduce_body,
        grid=(_N // _NB,),
        in_specs=[pl.BlockSpec((_DEG, _NB, _D), lambda i: (0, i, 0))],
        out_specs=pl.BlockSpec((_NB, _D), lambda i: (i, 0)),
        out_shape=jax.ShapeDtypeStruct((_N, _D), jnp.float32),
    )(ft)
    return out
